# Initial kernel scaffold; baseline (speedup 1.0000x reference)
import jax
import jax.numpy as jnp
from jax.experimental import pallas as pl


def kernel(x, params):
    raise NotImplementedError("write your pallas kernel here")


# speedup vs baseline: 3.7103x; 3.7103x over previous
import math

import jax
import jax.numpy as jnp
from jax.experimental import pallas as pl
from jax.experimental.pallas import tpu as pltpu

_DIM = 768
_FF = 1536
_E = 8
_B = 2
_S = 2048
_BS = _B * _S
_TB = 256
_NTB = _BS // _TB
_SB = _S // _TB
_NF = _S // 2 + 1
_NFP = 1152
_FB = 128
_NFB = _NFP // _FB
_TYPES = ['conv', 'fourier', 'mlp', 'conv', 'fourier', 'mlp', 'conv', 'fourier']
_FO = [i for i, t in enumerate(_TYPES) if t == 'fourier']
_SQRT1_2 = 0.7071067811865476


def _gelu(a):
    return 0.5 * a * (1.0 + jax.lax.erf(a * _SQRT1_2))


def _router_body(x_ref, w_ref, b_ref, wc_ref, sums_ref, aux_ref):
    i = pl.program_id(0)
    logits = jnp.dot(x_ref[...], w_ref[...],
                     preferred_element_type=jnp.float32) + b_ref[...]
    rl = logits[:, 0:_E]
    m = jnp.max(rl, axis=1, keepdims=True)
    ex = jnp.exp(rl - m)
    probs = ex / jnp.sum(ex, axis=1, keepdims=True)
    eidx = jax.lax.broadcasted_iota(jnp.int32, (_TB, _E), 1)
    m1 = jnp.max(probs, axis=1, keepdims=True)
    i1 = jnp.min(jnp.where(probs == m1, eidx, _E), axis=1, keepdims=True)
    oh1 = (eidx == i1).astype(jnp.float32)
    p2 = jnp.where(oh1 > 0, -1e30, probs)
    m2 = jnp.max(p2, axis=1, keepdims=True)
    i2 = jnp.min(jnp.where(p2 == m2, eidx, _E), axis=1, keepdims=True)
    oh2 = (eidx == i2).astype(jnp.float32)
    den = m1 + m2
    wz = (m1 / den) * oh1 + (m2 / den) * oh2
    gate = 1.0 / (1.0 + jnp.exp(-logits[:, _E:_E + 1]))
    pad = jnp.zeros((_TB, 128 - _E - 1), jnp.float32)
    wc_ref[...] = jnp.concatenate([wz, gate, pad], axis=1)
    zpad = jnp.zeros((_TB, 128 - _E), jnp.float32)
    cnt_row = jnp.sum(jnp.concatenate([oh1 + oh2, zpad], axis=1),
                      axis=0, keepdims=True)
    prob_row = jnp.sum(jnp.concatenate([probs, zpad], axis=1),
                       axis=0, keepdims=True)

    @pl.when(i == 0)
    def _():
        sums_ref[...] = jnp.zeros_like(sums_ref)

    sums_ref[0:1, :] += cnt_row
    sums_ref[1:2, :] += prob_row

    @pl.when(i == _NTB - 1)
    def _():
        load = sums_ref[0:1, :] / float(_BS * 2)
        imp = sums_ref[1:2, :] / float(_BS)
        aux_ref[...] = (float(_E) * jnp.sum(load * imp)).reshape(1, 1)


def _pt_body(x_ref, xm_ref, xp_ref, k_ref, w1_ref, b1_ref, w2_ref, b2_ref,
             wc_ref, out_ref, acc_ref):
    e = pl.program_id(0)
    i = pl.program_id(1)
    x = x_ref[...].astype(jnp.float32)
    k = k_ref[0]
    c = (k[0:1, :] * xm_ref[...].astype(jnp.float32) + k[1:2, :] * x
         + k[2:3, :] * xp_ref[...].astype(jnp.float32))
    h = (x + c).astype(jnp.bfloat16)
    a = jnp.dot(h, w1_ref[0], preferred_element_type=jnp.float32) + b1_ref[0]
    g = _gelu(a).astype(jnp.bfloat16)
    o = jnp.dot(g, w2_ref[0], preferred_element_type=jnp.float32) + b2_ref[0]
    col = jnp.where(e == 0, 8,
                    jnp.where(e == 1, 0,
                              jnp.where(e == 2, 2,
                                        jnp.where(e == 3, 3,
                                                  jnp.where(e == 4, 5, 6)))))
    lane = jax.lax.broadcasted_iota(jnp.int32, (_TB, 128), 1)
    w = jnp.sum(jnp.where(lane == col, wc_ref[...], 0.0), axis=1,
                keepdims=True)
    contrib = w * o
    base = i * _TB

    @pl.when(e == 0)
    def _():
        acc_ref[pl.ds(base, _TB), :] = contrib

    @pl.when(jnp.logical_and(e > 0, e < 5))
    def _():
        acc_ref[pl.ds(base, _TB), :] += contrib

    @pl.when(e == 5)
    def _():
        out_ref[...] = acc_ref[pl.ds(base, _TB), :] + contrib


def _ffwd_body(x_ref, c_ref, s_ref, cat_ref):
    x = x_ref[0].astype(jnp.bfloat16)
    dn = (((0,), (0,)), ((), ()))
    re = jax.lax.dot_general(c_ref[...], x, dn,
                             preferred_element_type=jnp.float32)
    im = jax.lax.dot_general(s_ref[...], x, dn,
                             preferred_element_type=jnp.float32)
    cat_ref[0] = jnp.concatenate([re, im], axis=1).astype(jnp.bfloat16)


def _fmlp_body(cat_ref, w1_ref, b1_ref, w2_ref, b2_ref, fo_ref):
    a = jnp.dot(cat_ref[0], w1_ref[0],
                preferred_element_type=jnp.float32) + b1_ref[0]
    g = _gelu(a).astype(jnp.bfloat16)
    fo_ref[0, 0] = (jnp.dot(g, w2_ref[0],
                            preferred_element_type=jnp.float32)
                    + b2_ref[0]).astype(jnp.bfloat16)


def _finv_body(fo_ref, ic_ref, is_ref, wc_ref, accin_ref, out_ref, acc_ref):
    e = pl.program_id(0)
    i = pl.program_id(1)
    f = fo_ref[0, 0]
    tre = jnp.dot(ic_ref[...], f[:, :_DIM],
                  preferred_element_type=jnp.float32)
    tim = jnp.dot(is_ref[...], f[:, _DIM:],
                  preferred_element_type=jnp.float32)
    t = tre + tim
    col = jnp.where(e == 0, 1, jnp.where(e == 1, 4, 7))
    lane = jax.lax.broadcasted_iota(jnp.int32, (_TB, 128), 1)
    w = jnp.sum(jnp.where(lane == col, wc_ref[...], 0.0), axis=1,
                keepdims=True)
    contrib = w * t
    base = i * _TB

    @pl.when(e == 0)
    def _():
        acc_ref[pl.ds(base, _TB), :] = accin_ref[...] + contrib

    @pl.when(e == 1)
    def _():
        acc_ref[pl.ds(base, _TB), :] += contrib

    @pl.when(e == 2)
    def _():
        out_ref[...] = acc_ref[pl.ds(base, _TB), :] + contrib


def kernel(x, params):
    xf = x.reshape(_BS, _DIM)
    xm1 = jnp.pad(x, ((0, 0), (1, 0), (0, 0)))[:, :_S].reshape(_BS, _DIM)
    xp1 = jnp.pad(x, ((0, 0), (0, 1), (0, 0)))[:, 1:].reshape(_BS, _DIM)

    wrg = jnp.concatenate(
        [params['router_W'], params['gate_W'],
         jnp.zeros((_DIM, 128 - _E - 1), jnp.float32)], axis=1)
    brg = jnp.concatenate(
        [params['router_b'], params['gate_b'],
         jnp.zeros((128 - _E - 1,), jnp.float32)]).reshape(1, 128)

    wcomb, _sums, aux_arr = pl.pallas_call(
        _router_body,
        grid=(_NTB,),
        in_specs=[
            pl.BlockSpec((_TB, _DIM), lambda i: (i, 0)),
            pl.BlockSpec((_DIM, 128), lambda i: (0, 0)),
            pl.BlockSpec((1, 128), lambda i: (0, 0)),
        ],
        out_specs=[
            pl.BlockSpec((_TB, 128), lambda i: (i, 0)),
            pl.BlockSpec((8, 128), lambda i: (0, 0)),
            pl.BlockSpec((1, 1), lambda i: (0, 0)),
        ],
        out_shape=[
            jax.ShapeDtypeStruct((_BS, 128), jnp.float32),
            jax.ShapeDtypeStruct((8, 128), jnp.float32),
            jax.ShapeDtypeStruct((1, 1), jnp.float32),
        ],
    )(xf, wrg, brg)

    pt = [None, 0, 2, 3, 5, 6]
    ks, w1s, b1s, w2s, b2s = [], [], [], [], []
    for eid in pt:
        if eid is None:
            ks.append(jnp.zeros((3, _DIM), jnp.float32))
            w1s.append(params['shared_W1']); b1s.append(params['shared_b1'])
            w2s.append(params['shared_W2']); b2s.append(params['shared_b2'])
        else:
            if _TYPES[eid] == 'conv':
                ks.append(jnp.transpose(params['e%d_conv' % eid][:, 0, :]))
            else:
                ks.append(jnp.zeros((3, _DIM), jnp.float32))
            w1s.append(params['e%d_W1' % eid]); b1s.append(params['e%d_b1' % eid])
            w2s.append(params['e%d_W2' % eid]); b2s.append(params['e%d_b2' % eid])
    K = jnp.stack(ks)
    W1 = jnp.stack(w1s).astype(jnp.bfloat16)
    B1 = jnp.stack(b1s).reshape(6, 1, _FF)
    W2 = jnp.stack(w2s).astype(jnp.bfloat16)
    B2 = jnp.stack(b2s).reshape(6, 1, _DIM)
    xb16 = xf.astype(jnp.bfloat16)
    xm16 = xm1.astype(jnp.bfloat16)
    xp16 = xp1.astype(jnp.bfloat16)

    acc_pt = pl.pallas_call(
        _pt_body,
        grid=(6, _NTB),
        in_specs=[
            pl.BlockSpec((_TB, _DIM), lambda e, i: (i, 0)),
            pl.BlockSpec((_TB, _DIM), lambda e, i: (i * (e % 2), 0)),
            pl.BlockSpec((_TB, _DIM), lambda e, i: (i * (e % 2), 0)),
            pl.BlockSpec((1, 3, _DIM), lambda e, i: (e, 0, 0)),
            pl.BlockSpec((1, _DIM, _FF), lambda e, i: (e, 0, 0)),
            pl.BlockSpec((1, 1, _FF), lambda e, i: (e, 0, 0)),
            pl.BlockSpec((1, _FF, _DIM), lambda e, i: (e, 0, 0)),
            pl.BlockSpec((1, 1, _DIM), lambda e, i: (e, 0, 0)),
            pl.BlockSpec((_TB, 128), lambda e, i: (i, 0)),
        ],
        out_specs=pl.BlockSpec((_TB, _DIM), lambda e, i: (i, 0)),
        out_shape=jax.ShapeDtypeStruct((_BS, _DIM), jnp.float32),
        scratch_shapes=[pltpu.VMEM((_BS, _DIM), jnp.float32)],
    )(xb16, xm16, xp16, K, W1, B1, W2, B2, wcomb)

    ti = jnp.arange(_S, dtype=jnp.int32)[:, None]
    ki = jnp.arange(_NFP, dtype=jnp.int32)[None, :]
    ang = (2.0 * math.pi / _S) * ((ti * ki) % _S).astype(jnp.float32)
    valid = (ki < _NF).astype(jnp.float32)
    cosm = jnp.cos(ang)
    sinm = jnp.sin(ang)
    Cf = (cosm * valid).astype(jnp.bfloat16)
    Sf = (-sinm * valid).astype(jnp.bfloat16)
    scale = jnp.where((ki == 0) | (ki == _S // 2), 1.0, 2.0) / _S * valid
    Ic = (scale * cosm).astype(jnp.bfloat16)
    Is = (-scale * sinm).astype(jnp.bfloat16)

    cat = pl.pallas_call(
        _ffwd_body,
        grid=(_B, _NFB),
        in_specs=[
            pl.BlockSpec((1, _S, _DIM), lambda b, kb: (b, 0, 0)),
            pl.BlockSpec((_S, _FB), lambda b, kb: (0, kb)),
            pl.BlockSpec((_S, _FB), lambda b, kb: (0, kb)),
        ],
        out_specs=pl.BlockSpec((1, _FB, 2 * _DIM), lambda b, kb: (b, kb, 0)),
        out_shape=jax.ShapeDtypeStruct((_B, _NFP, 2 * _DIM), jnp.bfloat16),
    )(x, Cf, Sf)

    W1f = jnp.stack([params['e%d_W1' % e] for e in _FO]).astype(jnp.bfloat16)
    B1f = jnp.stack([params['e%d_b1' % e] for e in _FO]).reshape(3, 1, _FF)
    W2f = jnp.stack([params['e%d_W2' % e] for e in _FO]).astype(jnp.bfloat16)
    B2f = jnp.stack([params['e%d_b2' % e] for e in _FO]).reshape(3, 1, 2 * _DIM)

    fo = pl.pallas_call(
        _fmlp_body,
        grid=(3, _B, _NFB),
        in_specs=[
            pl.BlockSpec((1, _FB, 2 * _DIM), lambda e, b, kb: (b, kb, 0)),
            pl.BlockSpec((1, 2 * _DIM, _FF), lambda e, b, kb: (e, 0, 0)),
            pl.BlockSpec((1, 1, _FF), lambda e, b, kb: (e, 0, 0)),
            pl.BlockSpec((1, _FF, 2 * _DIM), lambda e, b, kb: (e, 0, 0)),
            pl.BlockSpec((1, 1, 2 * _DIM), lambda e, b, kb: (e, 0, 0)),
        ],
        out_specs=pl.BlockSpec((1, 1, _FB, 2 * _DIM),
                               lambda e, b, kb: (e, b, kb, 0)),
        out_shape=jax.ShapeDtypeStruct((3, _B, _NFP, 2 * _DIM), jnp.bfloat16),
    )(cat, W1f, B1f, W2f, B2f)

    out_flat = pl.pallas_call(
        _finv_body,
        grid=(3, _NTB),
        in_specs=[
            pl.BlockSpec((1, 1, _NFP, 2 * _DIM),
                         lambda e, i: (e, i // _SB, 0, 0)),
            pl.BlockSpec((_TB, _NFP), lambda e, i: (i % _SB, 0)),
            pl.BlockSpec((_TB, _NFP), lambda e, i: (i % _SB, 0)),
            pl.BlockSpec((_TB, 128), lambda e, i: (i, 0)),
            pl.BlockSpec((_TB, _DIM), lambda e, i: (i, 0)),
        ],
        out_specs=pl.BlockSpec((_TB, _DIM), lambda e, i: (i, 0)),
        out_shape=jax.ShapeDtypeStruct((_BS, _DIM), jnp.float32),
        scratch_shapes=[pltpu.VMEM((_BS, _DIM), jnp.float32)],
    )(fo, Ic, Is, wcomb, acc_pt)

    out = out_flat.reshape(_B, _S, _DIM)
    aux = aux_arr[0, 0]
    return out, aux
